# baseline (speedup 1.0000x reference)
import functools

import jax
import jax.numpy as jnp
from jax import lax
from jax.experimental import pallas as pl
from jax.experimental.pallas import tpu as pltpu
from jax.experimental.pallas import tpu_sc as plsc

_TM = 256
_TN = 512
_TR = 1024
_GCH = 32
_CCH = 8


def _routing_body(n_e, x_ref, gate_wt_ref, ids_ref, ws_ref):
    e_pad = gate_wt_ref.shape[1]
    logits = lax.dot_general(
        x_ref[...], gate_wt_ref[...], (((1,), (0,)), ((), ())),
        preferred_element_type=jnp.float32,
    )
    lane = lax.broadcasted_iota(jnp.int32, logits.shape, 1)
    neg = jnp.float32(-jnp.inf)
    logits = jnp.where(lane < n_e, logits, neg)
    m1 = jnp.max(logits, axis=1)
    i1 = jnp.min(jnp.where(logits == m1[:, None], lane, e_pad), axis=1)
    l2 = jnp.where(lane == i1[:, None], neg, logits)
    m2 = jnp.max(l2, axis=1)
    i2 = jnp.min(jnp.where(l2 == m2[:, None], lane, e_pad), axis=1)
    w1 = 1.0 / (1.0 + jnp.exp(m2 - m1))
    ids_ref[0, :] = i1
    ids_ref[1, :] = i2
    ws_ref[0, :] = w1
    ws_ref[1, :] = 1.0 - w1


def _routing(x2, gate_w):
    t, d_in = x2.shape
    e = gate_w.shape[0]
    gate_wt = jnp.zeros((d_in, 128), jnp.float32).at[:, :e].set(gate_w.T)
    ids, ws = pl.pallas_call(
        functools.partial(_routing_body, e),
        grid=(t // _TR,),
        in_specs=[
            pl.BlockSpec((_TR, d_in), lambda i: (i, 0)),
            pl.BlockSpec((d_in, 128), lambda i: (0, 0)),
        ],
        out_specs=[
            pl.BlockSpec((2, _TR), lambda i: (0, i)),
            pl.BlockSpec((2, _TR), lambda i: (0, i)),
        ],
        out_shape=[
            jax.ShapeDtypeStruct((2, t), jnp.int32),
            jax.ShapeDtypeStruct((2, t), jnp.float32),
        ],
    )(x2, gate_wt)
    return ids, ws


def _dispatch(ids, n_e, n_tiles):
    t = ids.shape[1]
    tk = 2 * t
    ntot = n_tiles * _TM
    e_all = ids.reshape(tk)
    order = jnp.argsort(e_all, stable=True)
    e_sorted = e_all[order]
    counts = jnp.zeros((n_e,), jnp.int32).at[e_all].add(1)
    coff = jnp.concatenate([jnp.zeros((1,), jnp.int32), jnp.cumsum(counts)[:-1]])
    padded = ((counts + _TM - 1) // _TM) * _TM
    pcum = jnp.cumsum(padded)
    poff = jnp.concatenate([jnp.zeros((1,), jnp.int32), pcum[:-1]])
    rank = jnp.arange(tk, dtype=jnp.int32) - coff[e_sorted]
    ppos_sorted = poff[e_sorted] + rank
    ppos = jnp.zeros((tk,), jnp.int32).at[order].set(ppos_sorted)
    gidx = jnp.zeros((ntot,), jnp.int32).at[ppos_sorted].set(order % t)
    tile_expert = jnp.clip(
        jnp.searchsorted(pcum, jnp.arange(n_tiles, dtype=jnp.int32) * _TM,
                         side="right").astype(jnp.int32), 0, n_e - 1)
    return gidx, ppos.reshape(2, t), tile_expert


def _sc_gather(xp, gidx, ntot):
    info = plsc.get_sparse_core_info()
    nc, ns = info.num_cores, info.num_subcores
    nw = nc * ns
    per_w = ntot // nw
    nit = per_w // _GCH
    dp = xp.shape[1]
    mesh = plsc.VectorSubcoreMesh(core_axis_name="c", subcore_axis_name="s")

    @functools.partial(
        pl.kernel, mesh=mesh,
        out_type=jax.ShapeDtypeStruct((ntot, dp), jnp.int32),
        scratch_types=[
            pltpu.VMEM((_GCH,), jnp.int32),
            pltpu.VMEM((_GCH, dp), jnp.int32),
            pltpu.VMEM((_GCH, dp), jnp.int32),
            pltpu.SemaphoreType.DMA,
            pltpu.SemaphoreType.DMA,
            pltpu.SemaphoreType.DMA,
        ],
    )
    def k(xp_hbm, gidx_hbm, out_hbm, idx_v, rows0, rows1, gsem, ssem0, ssem1):
        wid = lax.axis_index("s") * nc + lax.axis_index("c")
        base = wid * per_w
        rows = (rows0, rows1)
        ssem = (ssem0, ssem1)

        def pair(g, carry):
            for b in range(2):
                it = 2 * g + b
                off = base + it * _GCH
                @pl.when(it >= 2)
                def _drain():
                    pltpu.make_async_copy(
                        rows[b], out_hbm.at[pl.ds(base, _GCH)], ssem[b]).wait()

                pltpu.sync_copy(gidx_hbm.at[pl.ds(off, _GCH)], idx_v)
                pltpu.async_copy(xp_hbm.at[idx_v], rows[b], gsem).wait()
                pltpu.async_copy(rows[b], out_hbm.at[pl.ds(off, _GCH)], ssem[b])
            return carry

        lax.fori_loop(0, nit // 2, pair, 0)
        pltpu.make_async_copy(rows0, out_hbm.at[pl.ds(base, _GCH)], ssem0).wait()
        pltpu.make_async_copy(rows1, out_hbm.at[pl.ds(base, _GCH)], ssem1).wait()

    return k(xp, gidx)


def _gmm_body(te_ref, x_ref, wt_ref, b_ref, wc_ref, o_ref):
    y = lax.dot_general(
        x_ref[...], wt_ref[0], (((1,), (0,)), ((), ())),
        preferred_element_type=jnp.float32,
    )
    o_ref[...] = (y + b_ref[0]) * wc_ref[...]


def _gmm(xs, tile_expert, wt, expert_b3, wcol, n_tiles, d_in, d_out):
    grid = (d_out // _TN, n_tiles)
    return pl.pallas_call(
        _gmm_body,
        grid_spec=pltpu.PrefetchScalarGridSpec(
            num_scalar_prefetch=1,
            grid=grid,
            in_specs=[
                pl.BlockSpec((_TM, d_in), lambda j, i, te: (i, 0)),
                pl.BlockSpec((1, d_in, _TN), lambda j, i, te: (te[i], 0, j)),
                pl.BlockSpec((1, 1, _TN), lambda j, i, te: (te[i], 0, j)),
                pl.BlockSpec((_TM, 1), lambda j, i, te: (i, 0)),
            ],
            out_specs=pl.BlockSpec((_TM, _TN), lambda j, i, te: (i, j)),
        ),
        out_shape=jax.ShapeDtypeStruct((n_tiles * _TM, d_out), jnp.float32),
        compiler_params=pltpu.CompilerParams(
            dimension_semantics=("arbitrary", "arbitrary"),
        ),
    )(tile_expert, xs, wt, expert_b3, wcol)


def _sc_combine(ys, p1, p2, t, d_out):
    info = plsc.get_sparse_core_info()
    nc, ns = info.num_cores, info.num_subcores
    nw = nc * ns
    per_w = t // nw
    mesh = plsc.VectorSubcoreMesh(core_axis_name="c", subcore_axis_name="s")
    nsub = d_out // 16

    nit = per_w // _CCH

    @functools.partial(
        pl.kernel, mesh=mesh,
        out_type=jax.ShapeDtypeStruct((t, d_out), jnp.float32),
        scratch_types=[
            pltpu.VMEM((_CCH,), jnp.int32),
            pltpu.VMEM((_CCH,), jnp.int32),
            pltpu.VMEM((_CCH,), jnp.int32),
            pltpu.VMEM((_CCH,), jnp.int32),
            pltpu.VMEM((_CCH, d_out), jnp.float32),
            pltpu.VMEM((_CCH, d_out), jnp.float32),
            pltpu.VMEM((_CCH, d_out), jnp.float32),
            pltpu.VMEM((_CCH, d_out), jnp.float32),
            pltpu.SemaphoreType.DMA,
            pltpu.SemaphoreType.DMA,
            pltpu.SemaphoreType.DMA,
            pltpu.SemaphoreType.DMA,
            pltpu.SemaphoreType.DMA,
            pltpu.SemaphoreType.DMA,
        ],
    )
    def k(ys_hbm, p1_hbm, p2_hbm, out_hbm, ia0, ia1, ib0, ib1, a0, b0, a1, b1,
          ga0, gb0, ga1, gb1, ss0, ss1):
        wid = lax.axis_index("s") * nc + lax.axis_index("c")
        base = wid * per_w
        bufa = (a0, a1)
        bufb = (b0, b1)
        gsa = (ga0, ga1)
        gsb = (gb0, gb1)
        ssem = (ss0, ss1)
        idxa = (ia0, ia1)
        idxb = (ib0, ib1)

        def issue(it, s):
            off = base + it * _CCH
            pltpu.sync_copy(p1_hbm.at[pl.ds(off, _CCH)], idxa[s])
            pltpu.async_copy(ys_hbm.at[idxa[s]], bufa[s], gsa[s])
            pltpu.sync_copy(p2_hbm.at[pl.ds(off, _CCH)], idxb[s])
            pltpu.async_copy(ys_hbm.at[idxb[s]], bufb[s], gsb[s])

        issue(0, 0)

        def pair(g, carry):
            for s in range(2):
                it = 2 * g + s
                nxt = it + 1
                ns_ = 1 - s
                @pl.when(nxt < nit)
                def _issue():
                    @pl.when(nxt >= 2)
                    def _drain():
                        pltpu.make_async_copy(
                            bufa[ns_], out_hbm.at[pl.ds(base, _CCH)],
                            ssem[ns_]).wait()

                    issue(nxt, ns_)

                pltpu.make_async_copy(
                    ys_hbm.at[idxa[s]], bufa[s], gsa[s]).wait()
                pltpu.make_async_copy(
                    ys_hbm.at[idxb[s]], bufb[s], gsb[s]).wait()

                def row(r, c2):
                    for c in range(nsub):
                        sl = pl.ds(c * 16, 16)
                        bufa[s][r, sl] = bufa[s][r, sl] + bufb[s][r, sl]
                    return c2

                lax.fori_loop(0, _CCH, row, 0)
                off = base + it * _CCH
                pltpu.async_copy(bufa[s], out_hbm.at[pl.ds(off, _CCH)], ssem[s])
            return carry

        lax.fori_loop(0, nit // 2, pair, 0)
        pltpu.make_async_copy(a0, out_hbm.at[pl.ds(base, _CCH)], ss0).wait()
        pltpu.make_async_copy(a1, out_hbm.at[pl.ds(base, _CCH)], ss1).wait()

    return k(ys, p1, p2)


@jax.jit
def kernel(inputs, gate_w, expert_w, expert_b):
    b, s, d_in = inputs.shape
    n_e, d_out, _ = expert_w.shape
    t = b * s
    n_tiles = (2 * t) // _TM + n_e
    ntot = n_tiles * _TM

    x2 = inputs.reshape(t, d_in)
    wt = jnp.swapaxes(expert_w, 1, 2).astype(jnp.bfloat16)
    expert_b3 = expert_b.reshape(n_e, 1, d_out)

    ids, ws = _routing(x2, gate_w)
    gidx, ppos, tile_expert = _dispatch(ids, n_e, n_tiles)
    wcol = jnp.zeros((ntot, 1), jnp.float32).at[ppos.reshape(-1), 0].set(
        ws.reshape(-1))

    xp = lax.bitcast_convert_type(
        x2.astype(jnp.bfloat16).reshape(t, d_in // 2, 2), jnp.int32)
    xsp = _sc_gather(xp, gidx, ntot)
    xs = lax.bitcast_convert_type(xsp, jnp.bfloat16).reshape(ntot, d_in)
    ys = _gmm(xs, tile_expert, wt, expert_b3, wcol, n_tiles, d_in, d_out)
    out = _sc_combine(ys, ppos[0], ppos[1], t, d_out)
    return out.reshape(b, s, d_out)


# speedup vs baseline: 1.9750x; 1.9750x over previous
import functools

import jax
import jax.numpy as jnp
from jax import lax
from jax.experimental import pallas as pl
from jax.experimental.pallas import tpu as pltpu
from jax.experimental.pallas import tpu_sc as plsc

_TM = 256
_TN = 512
_TR = 1024
_GCH = 32
_CCH = 8


def _routing_body(n_e, x_ref, gate_wt_ref, ids_ref, ws_ref, xp_ref):
    e_pad = gate_wt_ref.shape[1]
    logits = lax.dot_general(
        x_ref[...], gate_wt_ref[...], (((1,), (0,)), ((), ())),
        preferred_element_type=jnp.float32,
    )
    lane = lax.broadcasted_iota(jnp.int32, logits.shape, 1)
    neg = jnp.float32(-jnp.inf)
    logits = jnp.where(lane < n_e, logits, neg)
    m1 = jnp.max(logits, axis=1)
    i1 = jnp.min(jnp.where(logits == m1[:, None], lane, e_pad), axis=1)
    l2 = jnp.where(lane == i1[:, None], neg, logits)
    m2 = jnp.max(l2, axis=1)
    i2 = jnp.min(jnp.where(l2 == m2[:, None], lane, e_pad), axis=1)
    w1 = 1.0 / (1.0 + jnp.exp(m2 - m1))
    ids_ref[0, :] = i1
    ids_ref[1, :] = i2
    ws_ref[0, :] = w1
    ws_ref[1, :] = 1.0 - w1
    x = x_ref[...]
    half = x.shape[1] // 2
    lo = lax.bitcast_convert_type(
        x[:, :half].astype(jnp.bfloat16).astype(jnp.float32), jnp.uint32)
    hi = lax.bitcast_convert_type(
        x[:, half:].astype(jnp.bfloat16).astype(jnp.float32), jnp.uint32)
    xp_ref[...] = lax.bitcast_convert_type(
        (lo >> 16) | (hi & jnp.uint32(0xFFFF0000)), jnp.int32)


def _routing(x2, gate_w):
    t, d_in = x2.shape
    e = gate_w.shape[0]
    gate_wt = jnp.zeros((d_in, 128), jnp.float32).at[:, :e].set(gate_w.T)
    ids, ws, xp = pl.pallas_call(
        functools.partial(_routing_body, e),
        grid=(t // _TR,),
        in_specs=[
            pl.BlockSpec((_TR, d_in), lambda i: (i, 0)),
            pl.BlockSpec((d_in, 128), lambda i: (0, 0)),
        ],
        out_specs=[
            pl.BlockSpec((2, _TR), lambda i: (0, i)),
            pl.BlockSpec((2, _TR), lambda i: (0, i)),
            pl.BlockSpec((_TR, d_in // 2), lambda i: (i, 0)),
        ],
        out_shape=[
            jax.ShapeDtypeStruct((2, t), jnp.int32),
            jax.ShapeDtypeStruct((2, t), jnp.float32),
            jax.ShapeDtypeStruct((t, d_in // 2), jnp.int32),
        ],
    )(x2, gate_wt)
    return ids, ws, xp


def _dispatch(ids, n_e, n_tiles):
    t = ids.shape[1]
    tk = 2 * t
    ntot = n_tiles * _TM
    e_all = ids.reshape(tk)
    order = jnp.argsort(e_all, stable=True)
    e_sorted = e_all[order]
    counts = jnp.zeros((n_e,), jnp.int32).at[e_all].add(1)
    coff = jnp.concatenate([jnp.zeros((1,), jnp.int32), jnp.cumsum(counts)[:-1]])
    padded = ((counts + _TM - 1) // _TM) * _TM
    pcum = jnp.cumsum(padded)
    poff = jnp.concatenate([jnp.zeros((1,), jnp.int32), pcum[:-1]])
    rank = jnp.arange(tk, dtype=jnp.int32) - coff[e_sorted]
    ppos_sorted = poff[e_sorted] + rank
    ppos = jnp.zeros((tk,), jnp.int32).at[order].set(ppos_sorted)
    gidx = jnp.zeros((ntot,), jnp.int32).at[ppos_sorted].set(order % t)
    tile_expert = jnp.clip(
        jnp.searchsorted(pcum, jnp.arange(n_tiles, dtype=jnp.int32) * _TM,
                         side="right").astype(jnp.int32), 0, n_e - 1)
    return gidx, ppos.reshape(2, t), tile_expert


def _sc_gather(xp, gidx, ntot):
    info = plsc.get_sparse_core_info()
    nc, ns = info.num_cores, info.num_subcores
    nw = nc * ns
    per_w = ntot // nw
    nit = per_w // _GCH
    dp = xp.shape[1]
    mesh = plsc.VectorSubcoreMesh(core_axis_name="c", subcore_axis_name="s")

    @functools.partial(
        pl.kernel, mesh=mesh,
        out_type=jax.ShapeDtypeStruct((ntot, dp), jnp.int32),
        scratch_types=[
            pltpu.VMEM((_GCH,), jnp.int32),
            pltpu.VMEM((_GCH, dp), jnp.int32),
            pltpu.VMEM((_GCH, dp), jnp.int32),
            pltpu.SemaphoreType.DMA,
            pltpu.SemaphoreType.DMA,
            pltpu.SemaphoreType.DMA,
        ],
    )
    def k(xp_hbm, gidx_hbm, out_hbm, idx_v, rows0, rows1, gsem, ssem0, ssem1):
        wid = lax.axis_index("s") * nc + lax.axis_index("c")
        base = wid * per_w
        rows = (rows0, rows1)
        ssem = (ssem0, ssem1)

        def pair(g, carry):
            for b in range(2):
                it = 2 * g + b
                off = base + it * _GCH
                @pl.when(it >= 2)
                def _drain():
                    pltpu.make_async_copy(
                        rows[b], out_hbm.at[pl.ds(base, _GCH)], ssem[b]).wait()

                pltpu.sync_copy(gidx_hbm.at[pl.ds(off, _GCH)], idx_v)
                pltpu.async_copy(xp_hbm.at[idx_v], rows[b], gsem).wait()
                pltpu.async_copy(rows[b], out_hbm.at[pl.ds(off, _GCH)], ssem[b])
            return carry

        lax.fori_loop(0, nit // 2, pair, 0)
        pltpu.make_async_copy(rows0, out_hbm.at[pl.ds(base, _GCH)], ssem0).wait()
        pltpu.make_async_copy(rows1, out_hbm.at[pl.ds(base, _GCH)], ssem1).wait()

    return k(xp, gidx)


def _gmm_body(te_ref, x_ref, wt_ref, b_ref, wc_ref, o_ref):
    half = x_ref.shape[1]
    xu = lax.bitcast_convert_type(x_ref[...], jnp.uint32)
    lo = lax.bitcast_convert_type(xu << 16, jnp.float32).astype(jnp.bfloat16)
    hi = lax.bitcast_convert_type(
        xu & jnp.uint32(0xFFFF0000), jnp.float32).astype(jnp.bfloat16)
    dn = (((1,), (0,)), ((), ()))
    y = lax.dot_general(lo, wt_ref[0, :half, :], dn,
                        preferred_element_type=jnp.float32)
    y = y + lax.dot_general(hi, wt_ref[0, half:, :], dn,
                            preferred_element_type=jnp.float32)
    o_ref[...] = (y + b_ref[0]) * wc_ref[...]


def _gmm(xs, tile_expert, wt, expert_b3, wcol, n_tiles, d_in, d_out):
    grid = (d_out // _TN, n_tiles)
    return pl.pallas_call(
        _gmm_body,
        grid_spec=pltpu.PrefetchScalarGridSpec(
            num_scalar_prefetch=1,
            grid=grid,
            in_specs=[
                pl.BlockSpec((_TM, d_in // 2), lambda j, i, te: (i, 0)),
                pl.BlockSpec((1, d_in, _TN), lambda j, i, te: (te[i], 0, j)),
                pl.BlockSpec((1, 1, _TN), lambda j, i, te: (te[i], 0, j)),
                pl.BlockSpec((_TM, 1), lambda j, i, te: (i, 0)),
            ],
            out_specs=pl.BlockSpec((_TM, _TN), lambda j, i, te: (i, j)),
        ),
        out_shape=jax.ShapeDtypeStruct((n_tiles * _TM, d_out), jnp.float32),
        compiler_params=pltpu.CompilerParams(
            dimension_semantics=("arbitrary", "arbitrary"),
        ),
    )(tile_expert, xs, wt, expert_b3, wcol)


def _sc_combine(ys, p1, p2, t, d_out):
    info = plsc.get_sparse_core_info()
    nc, ns = info.num_cores, info.num_subcores
    nw = nc * ns
    per_w = t // nw
    mesh = plsc.VectorSubcoreMesh(core_axis_name="c", subcore_axis_name="s")
    nsub = d_out // 16

    nit = per_w // _CCH

    @functools.partial(
        pl.kernel, mesh=mesh,
        out_type=jax.ShapeDtypeStruct((t, d_out), jnp.float32),
        scratch_types=[
            pltpu.VMEM((_CCH,), jnp.int32),
            pltpu.VMEM((_CCH,), jnp.int32),
            pltpu.VMEM((_CCH,), jnp.int32),
            pltpu.VMEM((_CCH,), jnp.int32),
            pltpu.VMEM((_CCH, d_out), jnp.float32),
            pltpu.VMEM((_CCH, d_out), jnp.float32),
            pltpu.VMEM((_CCH, d_out), jnp.float32),
            pltpu.VMEM((_CCH, d_out), jnp.float32),
            pltpu.SemaphoreType.DMA,
            pltpu.SemaphoreType.DMA,
            pltpu.SemaphoreType.DMA,
            pltpu.SemaphoreType.DMA,
            pltpu.SemaphoreType.DMA,
            pltpu.SemaphoreType.DMA,
        ],
    )
    def k(ys_hbm, p1_hbm, p2_hbm, out_hbm, ia0, ia1, ib0, ib1, a0, b0, a1, b1,
          ga0, gb0, ga1, gb1, ss0, ss1):
        wid = lax.axis_index("s") * nc + lax.axis_index("c")
        base = wid * per_w
        bufa = (a0, a1)
        bufb = (b0, b1)
        gsa = (ga0, ga1)
        gsb = (gb0, gb1)
        ssem = (ss0, ss1)
        idxa = (ia0, ia1)
        idxb = (ib0, ib1)

        def issue(it, s):
            off = base + it * _CCH
            pltpu.sync_copy(p1_hbm.at[pl.ds(off, _CCH)], idxa[s])
            pltpu.async_copy(ys_hbm.at[idxa[s]], bufa[s], gsa[s])
            pltpu.sync_copy(p2_hbm.at[pl.ds(off, _CCH)], idxb[s])
            pltpu.async_copy(ys_hbm.at[idxb[s]], bufb[s], gsb[s])

        issue(0, 0)

        def pair(g, carry):
            for s in range(2):
                it = 2 * g + s
                nxt = it + 1
                ns_ = 1 - s
                @pl.when(nxt < nit)
                def _issue():
                    @pl.when(nxt >= 2)
                    def _drain():
                        pltpu.make_async_copy(
                            bufa[ns_], out_hbm.at[pl.ds(base, _CCH)],
                            ssem[ns_]).wait()

                    issue(nxt, ns_)

                pltpu.make_async_copy(
                    ys_hbm.at[idxa[s]], bufa[s], gsa[s]).wait()
                pltpu.make_async_copy(
                    ys_hbm.at[idxb[s]], bufb[s], gsb[s]).wait()

                def row(r, c2):
                    for c in range(nsub):
                        sl = pl.ds(c * 16, 16)
                        bufa[s][r, sl] = bufa[s][r, sl] + bufb[s][r, sl]
                    return c2

                lax.fori_loop(0, _CCH, row, 0)
                off = base + it * _CCH
                pltpu.async_copy(bufa[s], out_hbm.at[pl.ds(off, _CCH)], ssem[s])
            return carry

        lax.fori_loop(0, nit // 2, pair, 0)
        pltpu.make_async_copy(a0, out_hbm.at[pl.ds(base, _CCH)], ss0).wait()
        pltpu.make_async_copy(a1, out_hbm.at[pl.ds(base, _CCH)], ss1).wait()

    return k(ys, p1, p2)


@jax.jit
def kernel(inputs, gate_w, expert_w, expert_b):
    b, s, d_in = inputs.shape
    n_e, d_out, _ = expert_w.shape
    t = b * s
    n_tiles = (2 * t) // _TM + n_e
    ntot = n_tiles * _TM

    x2 = inputs.reshape(t, d_in)
    wt = jnp.swapaxes(expert_w, 1, 2).astype(jnp.bfloat16)
    expert_b3 = expert_b.reshape(n_e, 1, d_out)

    ids, ws, xp = _routing(x2, gate_w)
    gidx, ppos, tile_expert = _dispatch(ids, n_e, n_tiles)
    wcol = jnp.zeros((ntot, 1), jnp.float32).at[ppos.reshape(-1), 0].set(
        ws.reshape(-1))

    xsp = _sc_gather(xp, gidx, ntot)
    ys = _gmm(xsp, tile_expert, wt, expert_b3, wcol, n_tiles, d_in, d_out)
    out = _sc_combine(ys, ppos[0], ppos[1], t, d_out)
    return out.reshape(b, s, d_out)


# speedup vs baseline: 2.3624x; 1.1962x over previous
import functools

import jax
import jax.numpy as jnp
from jax import lax
from jax.experimental import pallas as pl
from jax.experimental.pallas import tpu as pltpu
from jax.experimental.pallas import tpu_sc as plsc

_TM = 256
_TN = 512
_TR = 1024
_GCH = 32
_CCH = 8


def _routing_body(n_e, x_ref, gate_wt_ref, ids_ref, ws_ref, xp_ref):
    e_pad = gate_wt_ref.shape[1]
    logits = lax.dot_general(
        x_ref[...], gate_wt_ref[...], (((1,), (0,)), ((), ())),
        preferred_element_type=jnp.float32,
    )
    lane = lax.broadcasted_iota(jnp.int32, logits.shape, 1)
    neg = jnp.float32(-jnp.inf)
    logits = jnp.where(lane < n_e, logits, neg)
    m1 = jnp.max(logits, axis=1)
    i1 = jnp.min(jnp.where(logits == m1[:, None], lane, e_pad), axis=1)
    l2 = jnp.where(lane == i1[:, None], neg, logits)
    m2 = jnp.max(l2, axis=1)
    i2 = jnp.min(jnp.where(l2 == m2[:, None], lane, e_pad), axis=1)
    w1 = 1.0 / (1.0 + jnp.exp(m2 - m1))
    ids_ref[0, :] = i1
    ids_ref[1, :] = i2
    ws_ref[0, :] = w1
    ws_ref[1, :] = 1.0 - w1
    x = x_ref[...]
    half = x.shape[1] // 2
    lo = lax.bitcast_convert_type(
        x[:, :half].astype(jnp.bfloat16).astype(jnp.float32), jnp.uint32)
    hi = lax.bitcast_convert_type(
        x[:, half:].astype(jnp.bfloat16).astype(jnp.float32), jnp.uint32)
    xp_ref[...] = lax.bitcast_convert_type(
        (lo >> 16) | (hi & jnp.uint32(0xFFFF0000)), jnp.int32)


def _routing(x2, gate_w):
    t, d_in = x2.shape
    e = gate_w.shape[0]
    gate_wt = jnp.zeros((d_in, 128), jnp.float32).at[:, :e].set(gate_w.T)
    ids, ws, xp = pl.pallas_call(
        functools.partial(_routing_body, e),
        grid=(t // _TR,),
        in_specs=[
            pl.BlockSpec((_TR, d_in), lambda i: (i, 0)),
            pl.BlockSpec((d_in, 128), lambda i: (0, 0)),
        ],
        out_specs=[
            pl.BlockSpec((2, _TR), lambda i: (0, i)),
            pl.BlockSpec((2, _TR), lambda i: (0, i)),
            pl.BlockSpec((_TR, d_in // 2), lambda i: (i, 0)),
        ],
        out_shape=[
            jax.ShapeDtypeStruct((2, t), jnp.int32),
            jax.ShapeDtypeStruct((2, t), jnp.float32),
            jax.ShapeDtypeStruct((t, d_in // 2), jnp.int32),
        ],
    )(x2, gate_wt)
    return ids, ws, xp


def _dispatch(ids, n_e, n_tiles):
    t = ids.shape[1]
    tk = 2 * t
    ntot = n_tiles * _TM
    e_all = ids.reshape(tk)
    order = jnp.argsort(e_all, stable=True)
    e_sorted = e_all[order]
    counts = jnp.zeros((n_e,), jnp.int32).at[e_all].add(1)
    coff = jnp.concatenate([jnp.zeros((1,), jnp.int32), jnp.cumsum(counts)[:-1]])
    padded = ((counts + _TM - 1) // _TM) * _TM
    pcum = jnp.cumsum(padded)
    poff = jnp.concatenate([jnp.zeros((1,), jnp.int32), pcum[:-1]])
    rank = jnp.arange(tk, dtype=jnp.int32) - coff[e_sorted]
    ppos_sorted = poff[e_sorted] + rank
    ppos = jnp.zeros((tk,), jnp.int32).at[order].set(ppos_sorted)
    gidx = jnp.zeros((ntot,), jnp.int32).at[ppos_sorted].set(order % t)
    tile_expert = jnp.clip(
        jnp.searchsorted(pcum, jnp.arange(n_tiles, dtype=jnp.int32) * _TM,
                         side="right").astype(jnp.int32), 0, n_e - 1)
    return gidx, ppos.reshape(2, t), tile_expert


def _sc_gather(xp, gidx, ntot):
    info = plsc.get_sparse_core_info()
    nc, ns = info.num_cores, info.num_subcores
    nw = nc * ns
    per_w = ntot // nw
    nit = per_w // _GCH
    dp = xp.shape[1]
    mesh = plsc.VectorSubcoreMesh(core_axis_name="c", subcore_axis_name="s")

    @functools.partial(
        pl.kernel, mesh=mesh,
        out_type=jax.ShapeDtypeStruct((ntot, dp), jnp.int32),
        scratch_types=[
            pltpu.VMEM((_GCH,), jnp.int32),
            pltpu.VMEM((_GCH, dp), jnp.int32),
            pltpu.VMEM((_GCH, dp), jnp.int32),
            pltpu.SemaphoreType.DMA,
            pltpu.SemaphoreType.DMA,
            pltpu.SemaphoreType.DMA,
        ],
    )
    def k(xp_hbm, gidx_hbm, out_hbm, idx_v, rows0, rows1, gsem, ssem0, ssem1):
        wid = lax.axis_index("s") * nc + lax.axis_index("c")
        base = wid * per_w
        rows = (rows0, rows1)
        ssem = (ssem0, ssem1)

        def pair(g, carry):
            for b in range(2):
                it = 2 * g + b
                off = base + it * _GCH
                @pl.when(it >= 2)
                def _drain():
                    pltpu.make_async_copy(
                        rows[b], out_hbm.at[pl.ds(base, _GCH)], ssem[b]).wait()

                pltpu.sync_copy(gidx_hbm.at[pl.ds(off, _GCH)], idx_v)
                pltpu.async_copy(xp_hbm.at[idx_v], rows[b], gsem).wait()
                pltpu.async_copy(rows[b], out_hbm.at[pl.ds(off, _GCH)], ssem[b])
            return carry

        lax.fori_loop(0, nit // 2, pair, 0)
        pltpu.make_async_copy(rows0, out_hbm.at[pl.ds(base, _GCH)], ssem0).wait()
        pltpu.make_async_copy(rows1, out_hbm.at[pl.ds(base, _GCH)], ssem1).wait()

    return k(xp, gidx)


def _gmm_body(te_ref, x_ref, w_ref, b_ref, wc_ref, o_ref):
    half = x_ref.shape[1]
    xu = lax.bitcast_convert_type(x_ref[...], jnp.uint32)
    lo = lax.bitcast_convert_type(xu << 16, jnp.float32).astype(jnp.bfloat16)
    hi = lax.bitcast_convert_type(
        xu & jnp.uint32(0xFFFF0000), jnp.float32).astype(jnp.bfloat16)
    w = w_ref[0].astype(jnp.bfloat16)
    dn = (((1,), (1,)), ((), ()))
    y = lax.dot_general(lo, w[:, :half], dn,
                        preferred_element_type=jnp.float32)
    y = y + lax.dot_general(hi, w[:, half:], dn,
                            preferred_element_type=jnp.float32)
    o_ref[...] = (y + b_ref[0]) * wc_ref[...]


def _gmm(xs, tile_expert, expert_w, expert_b3, wcol, n_tiles, d_in, d_out):
    grid = (d_out // _TN, n_tiles)
    return pl.pallas_call(
        _gmm_body,
        grid_spec=pltpu.PrefetchScalarGridSpec(
            num_scalar_prefetch=1,
            grid=grid,
            in_specs=[
                pl.BlockSpec((_TM, d_in // 2), lambda j, i, te: (i, 0)),
                pl.BlockSpec((1, _TN, d_in), lambda j, i, te: (te[i], j, 0)),
                pl.BlockSpec((1, 1, _TN), lambda j, i, te: (te[i], 0, j)),
                pl.BlockSpec((_TM, 1), lambda j, i, te: (i, 0)),
            ],
            out_specs=pl.BlockSpec((_TM, _TN), lambda j, i, te: (i, j)),
        ),
        out_shape=jax.ShapeDtypeStruct((n_tiles * _TM, d_out), jnp.float32),
        compiler_params=pltpu.CompilerParams(
            dimension_semantics=("arbitrary", "arbitrary"),
        ),
    )(tile_expert, xs, expert_w, expert_b3, wcol)


def _sc_combine(ys, p1, p2, t, d_out):
    info = plsc.get_sparse_core_info()
    nc, ns = info.num_cores, info.num_subcores
    nw = nc * ns
    per_w = t // nw
    mesh = plsc.VectorSubcoreMesh(core_axis_name="c", subcore_axis_name="s")
    nsub = d_out // 16

    nit = per_w // _CCH

    @functools.partial(
        pl.kernel, mesh=mesh,
        out_type=jax.ShapeDtypeStruct((t, d_out), jnp.float32),
        scratch_types=[
            pltpu.VMEM((_CCH,), jnp.int32),
            pltpu.VMEM((_CCH,), jnp.int32),
            pltpu.VMEM((_CCH,), jnp.int32),
            pltpu.VMEM((_CCH,), jnp.int32),
            pltpu.VMEM((_CCH, d_out), jnp.float32),
            pltpu.VMEM((_CCH, d_out), jnp.float32),
            pltpu.VMEM((_CCH, d_out), jnp.float32),
            pltpu.VMEM((_CCH, d_out), jnp.float32),
            pltpu.SemaphoreType.DMA,
            pltpu.SemaphoreType.DMA,
            pltpu.SemaphoreType.DMA,
            pltpu.SemaphoreType.DMA,
            pltpu.SemaphoreType.DMA,
            pltpu.SemaphoreType.DMA,
        ],
    )
    def k(ys_hbm, p1_hbm, p2_hbm, out_hbm, ia0, ia1, ib0, ib1, a0, b0, a1, b1,
          ga0, gb0, ga1, gb1, ss0, ss1):
        wid = lax.axis_index("s") * nc + lax.axis_index("c")
        base = wid * per_w
        bufa = (a0, a1)
        bufb = (b0, b1)
        gsa = (ga0, ga1)
        gsb = (gb0, gb1)
        ssem = (ss0, ss1)
        idxa = (ia0, ia1)
        idxb = (ib0, ib1)

        def issue(it, s):
            off = base + it * _CCH
            pltpu.sync_copy(p1_hbm.at[pl.ds(off, _CCH)], idxa[s])
            pltpu.async_copy(ys_hbm.at[idxa[s]], bufa[s], gsa[s])
            pltpu.sync_copy(p2_hbm.at[pl.ds(off, _CCH)], idxb[s])
            pltpu.async_copy(ys_hbm.at[idxb[s]], bufb[s], gsb[s])

        issue(0, 0)

        def pair(g, carry):
            for s in range(2):
                it = 2 * g + s
                nxt = it + 1
                ns_ = 1 - s
                @pl.when(nxt < nit)
                def _issue():
                    @pl.when(nxt >= 2)
                    def _drain():
                        pltpu.make_async_copy(
                            bufa[ns_], out_hbm.at[pl.ds(base, _CCH)],
                            ssem[ns_]).wait()

                    issue(nxt, ns_)

                pltpu.make_async_copy(
                    ys_hbm.at[idxa[s]], bufa[s], gsa[s]).wait()
                pltpu.make_async_copy(
                    ys_hbm.at[idxb[s]], bufb[s], gsb[s]).wait()

                def row(r, c2):
                    for c in range(nsub):
                        sl = pl.ds(c * 16, 16)
                        bufa[s][r, sl] = bufa[s][r, sl] + bufb[s][r, sl]
                    return c2

                lax.fori_loop(0, _CCH, row, 0)
                off = base + it * _CCH
                pltpu.async_copy(bufa[s], out_hbm.at[pl.ds(off, _CCH)], ssem[s])
            return carry

        lax.fori_loop(0, nit // 2, pair, 0)
        pltpu.make_async_copy(a0, out_hbm.at[pl.ds(base, _CCH)], ss0).wait()
        pltpu.make_async_copy(a1, out_hbm.at[pl.ds(base, _CCH)], ss1).wait()

    return k(ys, p1, p2)


@jax.jit
def kernel(inputs, gate_w, expert_w, expert_b):
    b, s, d_in = inputs.shape
    n_e, d_out, _ = expert_w.shape
    t = b * s
    n_tiles = (2 * t) // _TM + n_e
    ntot = n_tiles * _TM

    x2 = inputs.reshape(t, d_in)
    expert_b3 = expert_b.reshape(n_e, 1, d_out)

    ids, ws, xp = _routing(x2, gate_w)
    gidx, ppos, tile_expert = _dispatch(ids, n_e, n_tiles)
    wcol = jnp.zeros((ntot, 1), jnp.float32).at[ppos.reshape(-1), 0].set(
        ws.reshape(-1))

    xsp = _sc_gather(xp, gidx, ntot)
    ys = _gmm(xsp, tile_expert, expert_w, expert_b3, wcol, n_tiles, d_in, d_out)
    out = _sc_combine(ys, ppos[0], ppos[1], t, d_out)
    return out.reshape(b, s, d_out)


# speedup vs baseline: 2.5475x; 1.0783x over previous
import functools

import jax
import jax.numpy as jnp
from jax import lax
from jax.experimental import pallas as pl
from jax.experimental.pallas import tpu as pltpu
from jax.experimental.pallas import tpu_sc as plsc

_TM = 256
_TN = 512
_TR = 1024
_GCH = 32
_CCH = 8


def _routing_body(n_e, x_ref, gate_wt_ref, ids_ref, ws_ref, xp_ref):
    e_pad = gate_wt_ref.shape[1]
    logits = lax.dot_general(
        x_ref[...], gate_wt_ref[...], (((1,), (0,)), ((), ())),
        preferred_element_type=jnp.float32,
    )
    lane = lax.broadcasted_iota(jnp.int32, logits.shape, 1)
    neg = jnp.float32(-jnp.inf)
    logits = jnp.where(lane < n_e, logits, neg)
    m1 = jnp.max(logits, axis=1)
    i1 = jnp.min(jnp.where(logits == m1[:, None], lane, e_pad), axis=1)
    l2 = jnp.where(lane == i1[:, None], neg, logits)
    m2 = jnp.max(l2, axis=1)
    i2 = jnp.min(jnp.where(l2 == m2[:, None], lane, e_pad), axis=1)
    w1 = 1.0 / (1.0 + jnp.exp(m2 - m1))
    ids_ref[0, :] = i1
    ids_ref[1, :] = i2
    ws_ref[0, :] = w1
    ws_ref[1, :] = 1.0 - w1
    x = x_ref[...]
    half = x.shape[1] // 2
    lo = lax.bitcast_convert_type(
        x[:, :half].astype(jnp.bfloat16).astype(jnp.float32), jnp.uint32)
    hi = lax.bitcast_convert_type(
        x[:, half:].astype(jnp.bfloat16).astype(jnp.float32), jnp.uint32)
    xp_ref[...] = lax.bitcast_convert_type(
        (lo >> 16) | (hi & jnp.uint32(0xFFFF0000)), jnp.int32)


def _routing(x2, gate_w):
    t, d_in = x2.shape
    e = gate_w.shape[0]
    gate_wt = jnp.zeros((d_in, 128), jnp.float32).at[:, :e].set(gate_w.T)
    ids, ws, xp = pl.pallas_call(
        functools.partial(_routing_body, e),
        grid=(t // _TR,),
        in_specs=[
            pl.BlockSpec((_TR, d_in), lambda i: (i, 0)),
            pl.BlockSpec((d_in, 128), lambda i: (0, 0)),
        ],
        out_specs=[
            pl.BlockSpec((2, _TR), lambda i: (0, i)),
            pl.BlockSpec((2, _TR), lambda i: (0, i)),
            pl.BlockSpec((_TR, d_in // 2), lambda i: (i, 0)),
        ],
        out_shape=[
            jax.ShapeDtypeStruct((2, t), jnp.int32),
            jax.ShapeDtypeStruct((2, t), jnp.float32),
            jax.ShapeDtypeStruct((t, d_in // 2), jnp.int32),
        ],
    )(x2, gate_wt)
    return ids, ws, xp


def _dispatch(ids, ws, n_e, n_tiles):
    t = ids.shape[1]
    tk = 2 * t
    e_all = ids.reshape(tk)
    e_sorted, order, ws_sorted = lax.sort(
        (e_all, jnp.arange(tk, dtype=jnp.int32), ws.reshape(tk)),
        num_keys=1, is_stable=True)
    idx8 = jnp.arange(n_e, dtype=jnp.int32)
    coff = jnp.searchsorted(e_sorted, idx8, side="left").astype(jnp.int32)
    cend = jnp.searchsorted(e_sorted, idx8, side="right").astype(jnp.int32)
    counts = cend - coff
    padded = ((counts + _TM - 1) // _TM) * _TM
    pcum = jnp.cumsum(padded)
    poff = pcum - padded
    shift = poff - coff
    z8 = jnp.zeros((8,), jnp.int32)
    shift16 = jnp.concatenate([shift, z8])
    cend16 = jnp.concatenate([cend, z8])
    pcum16 = jnp.concatenate([pcum, z8])
    tile_expert = jnp.clip(
        jnp.searchsorted(pcum, jnp.arange(n_tiles, dtype=jnp.int32) * _TM,
                         side="right").astype(jnp.int32), 0, n_e - 1)
    pos_sorted = jnp.arange(tk, dtype=jnp.int32) + shift[e_sorted]
    inv = jnp.zeros((tk,), jnp.int32).at[order].set(pos_sorted)
    return (order, ws_sorted, shift16, cend16, pcum16, tile_expert,
            inv.reshape(2, t))


def _sc_gather(xp, order, ws_flat, shift16, cend16, pcum16, t, ntot):
    info = plsc.get_sparse_core_info()
    nc, ns = info.num_cores, info.num_subcores
    nw = nc * ns
    per_w = ntot // nw
    nit = per_w // _GCH
    tk = 2 * t
    dp = xp.shape[1]
    mesh = plsc.VectorSubcoreMesh(core_axis_name="c", subcore_axis_name="s")

    @functools.partial(
        pl.kernel, mesh=mesh,
        out_type=[
            jax.ShapeDtypeStruct((ntot, dp), jnp.int32),
            jax.ShapeDtypeStruct((ntot,), jnp.float32),
        ],
        scratch_types=[
            pltpu.VMEM((_GCH,), jnp.int32),
            pltpu.VMEM((per_w,), jnp.float32),
            pltpu.VMEM((_GCH, dp), jnp.int32),
            pltpu.VMEM((_GCH, dp), jnp.int32),
            pltpu.VMEM((ntot,), jnp.int32),
            pltpu.VMEM((ntot,), jnp.float32),
            pltpu.VMEM((16,), jnp.int32),
            pltpu.VMEM((16,), jnp.int32),
            pltpu.VMEM((16,), jnp.int32),
            pltpu.SemaphoreType.DMA,
            pltpu.SemaphoreType.DMA,
            pltpu.SemaphoreType.DMA,
        ],
    )
    def k(xp_hbm, order_hbm, ws_hbm, shift_hbm, cend_hbm, pcum_hbm,
          out_hbm, wcol_hbm,
          idx_v, wfull, rows0, rows1, order_v, ws_v, shift_v, cend_v,
          pcum_v, gsem, ssem0, ssem1):
        wid = lax.axis_index("s") * nc + lax.axis_index("c")
        base = wid * per_w
        rows = (rows0, rows1)
        ssem = (ssem0, ssem1)
        pltpu.sync_copy(order_hbm, order_v.at[pl.ds(0, tk)])
        pltpu.sync_copy(ws_hbm, ws_v.at[pl.ds(0, tk)])
        pltpu.sync_copy(shift_hbm, shift_v)
        pltpu.sync_copy(cend_hbm, cend_v)
        pltpu.sync_copy(pcum_hbm, pcum_v)
        lane = lax.iota(jnp.int32, 16)
        sh_vec = shift_v[...]
        ce_vec = cend_v[...]
        pc_vec = pcum_v[...]
        sh_s = [sh_vec[j] for j in range(8)]
        ce_s = [ce_vec[j] for j in range(8)]
        pc_s = [pc_vec[j] for j in range(8)]

        def chunk_meta(p0):
            e0 = jnp.int32(0)
            for j in range(7):
                e0 = e0 + (p0 >= pc_s[j]).astype(jnp.int32)
            sh0 = sh_s[0]
            ce0 = ce_s[0]
            for j in range(1, 8):
                sel = e0 == j
                sh0 = jnp.where(sel, sh_s[j], sh0)
                ce0 = jnp.where(sel, ce_s[j], ce0)
            return p0 - sh0, ce0

        def pair(g, carry):
            for b in range(2):
                it = 2 * g + b
                off = base + it * _GCH
                @pl.when(it >= 2)
                def _drain():
                    pltpu.make_async_copy(
                        rows[b], out_hbm.at[pl.ds(base, _GCH)], ssem[b]).wait()

                for sub in range(_GCH // 16):
                    p0 = off + sub * 16
                    r0, ce0 = chunk_meta(p0)
                    valid = r0 + lane < ce0
                    a = order_v[pl.ds(r0, 16)]
                    tok = jnp.where(valid, a & (t - 1), 0)
                    idx_v[pl.ds(sub * 16, 16)] = tok
                    w = ws_v[pl.ds(r0, 16)]
                    wfull[pl.ds(it * _GCH + sub * 16, 16)] = jnp.where(
                        valid, w, 0.0)

                pltpu.async_copy(xp_hbm.at[idx_v], rows[b], gsem).wait()
                pltpu.async_copy(rows[b], out_hbm.at[pl.ds(off, _GCH)], ssem[b])
            return carry

        lax.fori_loop(0, nit // 2, pair, 0)
        pltpu.sync_copy(wfull, wcol_hbm.at[pl.ds(base, per_w)])
        pltpu.make_async_copy(rows0, out_hbm.at[pl.ds(base, _GCH)], ssem0).wait()
        pltpu.make_async_copy(rows1, out_hbm.at[pl.ds(base, _GCH)], ssem1).wait()

    return k(xp, order, ws_flat, shift16, cend16, pcum16)


def _gmm_body(te_ref, x_ref, w_ref, b_ref, wc_ref, o_ref):
    half = x_ref.shape[1]
    xu = lax.bitcast_convert_type(x_ref[...], jnp.uint32)
    lo = lax.bitcast_convert_type(xu << 16, jnp.float32).astype(jnp.bfloat16)
    hi = lax.bitcast_convert_type(
        xu & jnp.uint32(0xFFFF0000), jnp.float32).astype(jnp.bfloat16)
    w = w_ref[0].astype(jnp.bfloat16)
    dn = (((1,), (1,)), ((), ()))
    y = lax.dot_general(lo, w[:, :half], dn,
                        preferred_element_type=jnp.float32)
    y = y + lax.dot_general(hi, w[:, half:], dn,
                            preferred_element_type=jnp.float32)
    o_ref[...] = (y + b_ref[0]) * wc_ref[...]


def _gmm(xs, tile_expert, expert_w, expert_b3, wcol, n_tiles, d_in, d_out):
    grid = (d_out // _TN, n_tiles)
    return pl.pallas_call(
        _gmm_body,
        grid_spec=pltpu.PrefetchScalarGridSpec(
            num_scalar_prefetch=1,
            grid=grid,
            in_specs=[
                pl.BlockSpec((_TM, d_in // 2), lambda j, i, te: (i, 0)),
                pl.BlockSpec((1, _TN, d_in), lambda j, i, te: (te[i], j, 0)),
                pl.BlockSpec((1, 1, _TN), lambda j, i, te: (te[i], 0, j)),
                pl.BlockSpec((_TM, 1), lambda j, i, te: (i, 0)),
            ],
            out_specs=pl.BlockSpec((_TM, _TN), lambda j, i, te: (i, j)),
        ),
        out_shape=jax.ShapeDtypeStruct((n_tiles * _TM, d_out), jnp.float32),
        compiler_params=pltpu.CompilerParams(
            dimension_semantics=("arbitrary", "arbitrary"),
        ),
    )(tile_expert, xs, expert_w, expert_b3, wcol)


def _sc_combine(ys, inv, t, d_out):
    info = plsc.get_sparse_core_info()
    nc, ns = info.num_cores, info.num_subcores
    nw = nc * ns
    per_w = t // nw
    mesh = plsc.VectorSubcoreMesh(core_axis_name="c", subcore_axis_name="s")
    nsub = d_out // 16

    nit = per_w // _CCH

    @functools.partial(
        pl.kernel, mesh=mesh,
        out_type=jax.ShapeDtypeStruct((t, d_out), jnp.float32),
        scratch_types=[
            pltpu.VMEM((_CCH,), jnp.int32),
            pltpu.VMEM((_CCH,), jnp.int32),
            pltpu.VMEM((_CCH,), jnp.int32),
            pltpu.VMEM((_CCH,), jnp.int32),
            pltpu.VMEM((_CCH, d_out), jnp.float32),
            pltpu.VMEM((_CCH, d_out), jnp.float32),
            pltpu.VMEM((_CCH, d_out), jnp.float32),
            pltpu.VMEM((_CCH, d_out), jnp.float32),
            pltpu.SemaphoreType.DMA,
            pltpu.SemaphoreType.DMA,
            pltpu.SemaphoreType.DMA,
            pltpu.SemaphoreType.DMA,
            pltpu.SemaphoreType.DMA,
            pltpu.SemaphoreType.DMA,
        ],
    )
    def k(ys_hbm, inv_hbm, out_hbm, ia0, ia1, ib0, ib1, a0, b0, a1, b1,
          ga0, gb0, ga1, gb1, ss0, ss1):
        wid = lax.axis_index("s") * nc + lax.axis_index("c")
        base = wid * per_w
        bufa = (a0, a1)
        bufb = (b0, b1)
        gsa = (ga0, ga1)
        gsb = (gb0, gb1)
        ssem = (ss0, ss1)
        idxa = (ia0, ia1)
        idxb = (ib0, ib1)

        def issue(it, s):
            off = base + it * _CCH
            pltpu.sync_copy(inv_hbm.at[0, pl.ds(off, _CCH)], idxa[s])
            pltpu.async_copy(ys_hbm.at[idxa[s]], bufa[s], gsa[s])
            pltpu.sync_copy(inv_hbm.at[1, pl.ds(off, _CCH)], idxb[s])
            pltpu.async_copy(ys_hbm.at[idxb[s]], bufb[s], gsb[s])

        issue(0, 0)

        def pair(g, carry):
            for s in range(2):
                it = 2 * g + s
                nxt = it + 1
                ns_ = 1 - s
                @pl.when(nxt < nit)
                def _issue():
                    @pl.when(nxt >= 2)
                    def _drain():
                        pltpu.make_async_copy(
                            bufa[ns_], out_hbm.at[pl.ds(base, _CCH)],
                            ssem[ns_]).wait()

                    issue(nxt, ns_)

                pltpu.make_async_copy(
                    ys_hbm.at[idxa[s]], bufa[s], gsa[s]).wait()
                pltpu.make_async_copy(
                    ys_hbm.at[idxb[s]], bufb[s], gsb[s]).wait()

                def row(r, c2):
                    for c in range(nsub):
                        sl = pl.ds(c * 16, 16)
                        bufa[s][r, sl] = bufa[s][r, sl] + bufb[s][r, sl]
                    return c2

                lax.fori_loop(0, _CCH, row, 0)
                off = base + it * _CCH
                pltpu.async_copy(bufa[s], out_hbm.at[pl.ds(off, _CCH)], ssem[s])
            return carry

        lax.fori_loop(0, nit // 2, pair, 0)
        pltpu.make_async_copy(a0, out_hbm.at[pl.ds(base, _CCH)], ss0).wait()
        pltpu.make_async_copy(a1, out_hbm.at[pl.ds(base, _CCH)], ss1).wait()

    return k(ys, inv)


@jax.jit
def kernel(inputs, gate_w, expert_w, expert_b):
    b, s, d_in = inputs.shape
    n_e, d_out, _ = expert_w.shape
    t = b * s
    n_tiles = (2 * t) // _TM + n_e
    ntot = n_tiles * _TM

    x2 = inputs.reshape(t, d_in)
    expert_b3 = expert_b.reshape(n_e, 1, d_out)

    ids, ws, xp = _routing(x2, gate_w)
    order, ws_sorted, shift16, cend16, pcum16, tile_expert, inv = _dispatch(
        ids, ws, n_e, n_tiles)
    xsp, wcol_flat = _sc_gather(
        xp, order, ws_sorted, shift16, cend16, pcum16, t, ntot)
    ys = _gmm(xsp, tile_expert, expert_w, expert_b3,
              wcol_flat.reshape(ntot, 1), n_tiles, d_in, d_out)
    out = _sc_combine(ys, inv, t, d_out)
    return out.reshape(b, s, d_out)


# speedup vs baseline: 2.9542x; 1.1597x over previous
import functools

import jax
import jax.numpy as jnp
from jax import lax
from jax.experimental import pallas as pl
from jax.experimental.pallas import tpu as pltpu
from jax.experimental.pallas import tpu_sc as plsc

_TM = 256
_TN = 1024
_TR = 1024
_GCH = 32
_CCH = 8


def _routing_body(n_e, x_ref, gate_wt_ref, ids_ref, ws_ref, xp_ref):
    e_pad = gate_wt_ref.shape[1]
    logits = lax.dot_general(
        x_ref[...], gate_wt_ref[...], (((1,), (0,)), ((), ())),
        preferred_element_type=jnp.float32,
    )
    lane = lax.broadcasted_iota(jnp.int32, logits.shape, 1)
    neg = jnp.float32(-jnp.inf)
    logits = jnp.where(lane < n_e, logits, neg)
    m1 = jnp.max(logits, axis=1)
    i1 = jnp.min(jnp.where(logits == m1[:, None], lane, e_pad), axis=1)
    l2 = jnp.where(lane == i1[:, None], neg, logits)
    m2 = jnp.max(l2, axis=1)
    i2 = jnp.min(jnp.where(l2 == m2[:, None], lane, e_pad), axis=1)
    w1 = 1.0 / (1.0 + jnp.exp(m2 - m1))
    ids_ref[0, :] = i1
    ids_ref[1, :] = i2
    ws_ref[0, :] = w1
    ws_ref[1, :] = 1.0 - w1
    x = x_ref[...]
    half = x.shape[1] // 2
    lo = lax.bitcast_convert_type(
        x[:, :half].astype(jnp.bfloat16).astype(jnp.float32), jnp.uint32)
    hi = lax.bitcast_convert_type(
        x[:, half:].astype(jnp.bfloat16).astype(jnp.float32), jnp.uint32)
    xp_ref[...] = lax.bitcast_convert_type(
        (lo >> 16) | (hi & jnp.uint32(0xFFFF0000)), jnp.int32)


def _routing(x2, gate_w):
    t, d_in = x2.shape
    e = gate_w.shape[0]
    gate_wt = jnp.zeros((d_in, 128), jnp.float32).at[:, :e].set(gate_w.T)
    ids, ws, xp = pl.pallas_call(
        functools.partial(_routing_body, e),
        grid=(t // _TR,),
        in_specs=[
            pl.BlockSpec((_TR, d_in), lambda i: (i, 0)),
            pl.BlockSpec((d_in, 128), lambda i: (0, 0)),
        ],
        out_specs=[
            pl.BlockSpec((2, _TR), lambda i: (0, i)),
            pl.BlockSpec((2, _TR), lambda i: (0, i)),
            pl.BlockSpec((_TR, d_in // 2), lambda i: (i, 0)),
        ],
        out_shape=[
            jax.ShapeDtypeStruct((2, t), jnp.int32),
            jax.ShapeDtypeStruct((2, t), jnp.float32),
            jax.ShapeDtypeStruct((t, d_in // 2), jnp.int32),
        ],
    )(x2, gate_wt)
    return ids, ws, xp


def _dispatch(ids, ws, n_e, n_tiles):
    t = ids.shape[1]
    tk = 2 * t
    e_all = ids.reshape(tk)
    e_sorted, order, ws_sorted = lax.sort(
        (e_all, jnp.arange(tk, dtype=jnp.int32), ws.reshape(tk)),
        num_keys=1, is_stable=True)
    idx8 = jnp.arange(n_e, dtype=jnp.int32)
    coff = jnp.searchsorted(e_sorted, idx8, side="left").astype(jnp.int32)
    cend = jnp.searchsorted(e_sorted, idx8, side="right").astype(jnp.int32)
    counts = cend - coff
    padded = ((counts + _TM - 1) // _TM) * _TM
    pcum = jnp.cumsum(padded)
    poff = pcum - padded
    shift = poff - coff
    z8 = jnp.zeros((8,), jnp.int32)
    shift16 = jnp.concatenate([shift, z8])
    cend16 = jnp.concatenate([cend, z8])
    pcum16 = jnp.concatenate([pcum, z8])
    tile_expert = jnp.clip(
        jnp.searchsorted(pcum, jnp.arange(n_tiles, dtype=jnp.int32) * _TM,
                         side="right").astype(jnp.int32), 0, n_e - 1)
    pos_sorted = jnp.arange(tk, dtype=jnp.int32) + shift[e_sorted]
    inv = jnp.zeros((tk,), jnp.int32).at[order].set(pos_sorted)
    return (order, ws_sorted, shift16, cend16, pcum16, tile_expert,
            inv.reshape(2, t))


def _sc_gather(xp, order, ws_flat, shift16, cend16, pcum16, t, ntot):
    info = plsc.get_sparse_core_info()
    nc, ns = info.num_cores, info.num_subcores
    nw = nc * ns
    per_w = ntot // nw
    nit = per_w // _GCH
    tk = 2 * t
    dp = xp.shape[1]
    mesh = plsc.VectorSubcoreMesh(core_axis_name="c", subcore_axis_name="s")

    @functools.partial(
        pl.kernel, mesh=mesh,
        out_type=[
            jax.ShapeDtypeStruct((ntot, dp), jnp.int32),
            jax.ShapeDtypeStruct((ntot,), jnp.float32),
        ],
        scratch_types=[
            pltpu.VMEM((_GCH,), jnp.int32),
            pltpu.VMEM((per_w,), jnp.float32),
            pltpu.VMEM((_GCH, dp), jnp.int32),
            pltpu.VMEM((_GCH, dp), jnp.int32),
            pltpu.VMEM((ntot,), jnp.int32),
            pltpu.VMEM((ntot,), jnp.float32),
            pltpu.VMEM((16,), jnp.int32),
            pltpu.VMEM((16,), jnp.int32),
            pltpu.VMEM((16,), jnp.int32),
            pltpu.SemaphoreType.DMA,
            pltpu.SemaphoreType.DMA,
            pltpu.SemaphoreType.DMA,
        ],
    )
    def k(xp_hbm, order_hbm, ws_hbm, shift_hbm, cend_hbm, pcum_hbm,
          out_hbm, wcol_hbm,
          idx_v, wfull, rows0, rows1, order_v, ws_v, shift_v, cend_v,
          pcum_v, gsem, ssem0, ssem1):
        wid = lax.axis_index("s") * nc + lax.axis_index("c")
        base = wid * per_w
        rows = (rows0, rows1)
        ssem = (ssem0, ssem1)
        pltpu.sync_copy(order_hbm, order_v.at[pl.ds(0, tk)])
        pltpu.sync_copy(ws_hbm, ws_v.at[pl.ds(0, tk)])
        pltpu.sync_copy(shift_hbm, shift_v)
        pltpu.sync_copy(cend_hbm, cend_v)
        pltpu.sync_copy(pcum_hbm, pcum_v)
        lane = lax.iota(jnp.int32, 16)
        sh_vec = shift_v[...]
        ce_vec = cend_v[...]
        pc_vec = pcum_v[...]
        sh_s = [sh_vec[j] for j in range(8)]
        ce_s = [ce_vec[j] for j in range(8)]
        pc_s = [pc_vec[j] for j in range(8)]

        def chunk_meta(p0):
            e0 = jnp.int32(0)
            for j in range(7):
                e0 = e0 + (p0 >= pc_s[j]).astype(jnp.int32)
            sh0 = sh_s[0]
            ce0 = ce_s[0]
            for j in range(1, 8):
                sel = e0 == j
                sh0 = jnp.where(sel, sh_s[j], sh0)
                ce0 = jnp.where(sel, ce_s[j], ce0)
            return p0 - sh0, ce0

        def pair(g, carry):
            for b in range(2):
                it = 2 * g + b
                off = base + it * _GCH
                @pl.when(it >= 2)
                def _drain():
                    pltpu.make_async_copy(
                        rows[b], out_hbm.at[pl.ds(base, _GCH)], ssem[b]).wait()

                for sub in range(_GCH // 16):
                    p0 = off + sub * 16
                    r0, ce0 = chunk_meta(p0)
                    valid = r0 + lane < ce0
                    a = order_v[pl.ds(r0, 16)]
                    tok = jnp.where(valid, a & (t - 1), 0)
                    idx_v[pl.ds(sub * 16, 16)] = tok
                    w = ws_v[pl.ds(r0, 16)]
                    wfull[pl.ds(it * _GCH + sub * 16, 16)] = jnp.where(
                        valid, w, 0.0)

                pltpu.async_copy(xp_hbm.at[idx_v], rows[b], gsem).wait()
                pltpu.async_copy(rows[b], out_hbm.at[pl.ds(off, _GCH)], ssem[b])
            return carry

        lax.fori_loop(0, nit // 2, pair, 0)
        pltpu.sync_copy(wfull, wcol_hbm.at[pl.ds(base, per_w)])
        pltpu.make_async_copy(rows0, out_hbm.at[pl.ds(base, _GCH)], ssem0).wait()
        pltpu.make_async_copy(rows1, out_hbm.at[pl.ds(base, _GCH)], ssem1).wait()

    return k(xp, order, ws_flat, shift16, cend16, pcum16)


def _gmm_body(te_ref, x_ref, w_ref, b_ref, wc_ref, o_ref):
    half = x_ref.shape[1]
    xu = lax.bitcast_convert_type(x_ref[...], jnp.uint32)
    lo = lax.bitcast_convert_type(xu << 16, jnp.float32).astype(jnp.bfloat16)
    hi = lax.bitcast_convert_type(
        xu & jnp.uint32(0xFFFF0000), jnp.float32).astype(jnp.bfloat16)
    w = w_ref[0].astype(jnp.bfloat16)
    dn = (((1,), (1,)), ((), ()))
    y = lax.dot_general(lo, w[:, :half], dn,
                        preferred_element_type=jnp.float32)
    y = y + lax.dot_general(hi, w[:, half:], dn,
                            preferred_element_type=jnp.float32)
    o_ref[...] = (y + b_ref[0]) * wc_ref[...]


def _gmm(xs, tile_expert, expert_w, expert_b3, wcol, n_tiles, d_in, d_out):
    grid = (d_out // _TN, n_tiles)
    return pl.pallas_call(
        _gmm_body,
        grid_spec=pltpu.PrefetchScalarGridSpec(
            num_scalar_prefetch=1,
            grid=grid,
            in_specs=[
                pl.BlockSpec((_TM, d_in // 2), lambda j, i, te: (i, 0)),
                pl.BlockSpec((1, _TN, d_in), lambda j, i, te: (te[i], j, 0)),
                pl.BlockSpec((1, 1, _TN), lambda j, i, te: (te[i], 0, j)),
                pl.BlockSpec((_TM, 1), lambda j, i, te: (i, 0)),
            ],
            out_specs=pl.BlockSpec((_TM, _TN), lambda j, i, te: (i, j)),
        ),
        out_shape=jax.ShapeDtypeStruct((n_tiles * _TM, d_out), jnp.float32),
        compiler_params=pltpu.CompilerParams(
            dimension_semantics=("arbitrary", "arbitrary"),
        ),
    )(tile_expert, xs, expert_w, expert_b3, wcol)


def _sc_combine(ys, inv, t, d_out):
    info = plsc.get_sparse_core_info()
    nc, ns = info.num_cores, info.num_subcores
    nw = nc * ns
    per_w = t // nw
    mesh = plsc.VectorSubcoreMesh(core_axis_name="c", subcore_axis_name="s")
    nsub = d_out // 16

    nit = per_w // _CCH

    @functools.partial(
        pl.kernel, mesh=mesh,
        out_type=jax.ShapeDtypeStruct((t, d_out), jnp.float32),
        scratch_types=[
            pltpu.VMEM((_CCH,), jnp.int32),
            pltpu.VMEM((_CCH,), jnp.int32),
            pltpu.VMEM((_CCH,), jnp.int32),
            pltpu.VMEM((_CCH,), jnp.int32),
            pltpu.VMEM((_CCH, d_out), jnp.float32),
            pltpu.VMEM((_CCH, d_out), jnp.float32),
            pltpu.VMEM((_CCH, d_out), jnp.float32),
            pltpu.VMEM((_CCH, d_out), jnp.float32),
            pltpu.SemaphoreType.DMA,
            pltpu.SemaphoreType.DMA,
            pltpu.SemaphoreType.DMA,
            pltpu.SemaphoreType.DMA,
            pltpu.SemaphoreType.DMA,
            pltpu.SemaphoreType.DMA,
        ],
    )
    def k(ys_hbm, inv_hbm, out_hbm, ia0, ia1, ib0, ib1, a0, b0, a1, b1,
          ga0, gb0, ga1, gb1, ss0, ss1):
        wid = lax.axis_index("s") * nc + lax.axis_index("c")
        base = wid * per_w
        bufa = (a0, a1)
        bufb = (b0, b1)
        gsa = (ga0, ga1)
        gsb = (gb0, gb1)
        ssem = (ss0, ss1)
        idxa = (ia0, ia1)
        idxb = (ib0, ib1)

        def issue(it, s):
            off = base + it * _CCH
            pltpu.sync_copy(inv_hbm.at[0, pl.ds(off, _CCH)], idxa[s])
            pltpu.async_copy(ys_hbm.at[idxa[s]], bufa[s], gsa[s])
            pltpu.sync_copy(inv_hbm.at[1, pl.ds(off, _CCH)], idxb[s])
            pltpu.async_copy(ys_hbm.at[idxb[s]], bufb[s], gsb[s])

        issue(0, 0)

        def pair(g, carry):
            for s in range(2):
                it = 2 * g + s
                nxt = it + 1
                ns_ = 1 - s
                @pl.when(nxt < nit)
                def _issue():
                    @pl.when(nxt >= 2)
                    def _drain():
                        pltpu.make_async_copy(
                            bufa[ns_], out_hbm.at[pl.ds(base, _CCH)],
                            ssem[ns_]).wait()

                    issue(nxt, ns_)

                pltpu.make_async_copy(
                    ys_hbm.at[idxa[s]], bufa[s], gsa[s]).wait()
                pltpu.make_async_copy(
                    ys_hbm.at[idxb[s]], bufb[s], gsb[s]).wait()

                def row(r, c2):
                    for c in range(nsub):
                        sl = pl.ds(c * 16, 16)
                        bufa[s][r, sl] = bufa[s][r, sl] + bufb[s][r, sl]
                    return c2

                lax.fori_loop(0, _CCH, row, 0)
                off = base + it * _CCH
                pltpu.async_copy(bufa[s], out_hbm.at[pl.ds(off, _CCH)], ssem[s])
            return carry

        lax.fori_loop(0, nit // 2, pair, 0)
        pltpu.make_async_copy(a0, out_hbm.at[pl.ds(base, _CCH)], ss0).wait()
        pltpu.make_async_copy(a1, out_hbm.at[pl.ds(base, _CCH)], ss1).wait()

    return k(ys, inv)


@jax.jit
def kernel(inputs, gate_w, expert_w, expert_b):
    b, s, d_in = inputs.shape
    n_e, d_out, _ = expert_w.shape
    t = b * s
    n_tiles = (2 * t) // _TM + n_e
    ntot = n_tiles * _TM

    x2 = inputs.reshape(t, d_in)
    expert_b3 = expert_b.reshape(n_e, 1, d_out)

    ids, ws, xp = _routing(x2, gate_w)
    order, ws_sorted, shift16, cend16, pcum16, tile_expert, inv = _dispatch(
        ids, ws, n_e, n_tiles)
    xsp, wcol_flat = _sc_gather(
        xp, order, ws_sorted, shift16, cend16, pcum16, t, ntot)
    ys = _gmm(xsp, tile_expert, expert_w, expert_b3,
              wcol_flat.reshape(ntot, 1), n_tiles, d_in, d_out)
    out = _sc_combine(ys, inv, t, d_out)
    return out.reshape(b, s, d_out)


# speedup vs baseline: 3.1530x; 1.0673x over previous
import functools

import jax
import jax.numpy as jnp
from jax import lax
from jax.experimental import pallas as pl
from jax.experimental.pallas import tpu as pltpu
from jax.experimental.pallas import tpu_sc as plsc

_TM = 256
_TN = 2048
_TR = 1024
_GCH = 32
_CCH = 8


def _routing_body(n_e, x_ref, gate_wt_ref, ids_ref, ws_ref, xp_ref):
    e_pad = gate_wt_ref.shape[1]
    logits = lax.dot_general(
        x_ref[...], gate_wt_ref[...], (((1,), (0,)), ((), ())),
        preferred_element_type=jnp.float32,
    )
    lane = lax.broadcasted_iota(jnp.int32, logits.shape, 1)
    neg = jnp.float32(-jnp.inf)
    logits = jnp.where(lane < n_e, logits, neg)
    m1 = jnp.max(logits, axis=1)
    i1 = jnp.min(jnp.where(logits == m1[:, None], lane, e_pad), axis=1)
    l2 = jnp.where(lane == i1[:, None], neg, logits)
    m2 = jnp.max(l2, axis=1)
    i2 = jnp.min(jnp.where(l2 == m2[:, None], lane, e_pad), axis=1)
    w1 = 1.0 / (1.0 + jnp.exp(m2 - m1))
    ids_ref[0, :] = i1
    ids_ref[1, :] = i2
    ws_ref[0, :] = w1
    ws_ref[1, :] = 1.0 - w1
    x = x_ref[...]
    half = x.shape[1] // 2
    lo = lax.bitcast_convert_type(
        x[:, :half].astype(jnp.bfloat16).astype(jnp.float32), jnp.uint32)
    hi = lax.bitcast_convert_type(
        x[:, half:].astype(jnp.bfloat16).astype(jnp.float32), jnp.uint32)
    xp_ref[...] = lax.bitcast_convert_type(
        (lo >> 16) | (hi & jnp.uint32(0xFFFF0000)), jnp.int32)


def _routing(x2, gate_w):
    t, d_in = x2.shape
    e = gate_w.shape[0]
    gate_wt = jnp.zeros((d_in, 128), jnp.float32).at[:, :e].set(gate_w.T)
    ids, ws, xp = pl.pallas_call(
        functools.partial(_routing_body, e),
        grid=(t // _TR,),
        in_specs=[
            pl.BlockSpec((_TR, d_in), lambda i: (i, 0)),
            pl.BlockSpec((d_in, 128), lambda i: (0, 0)),
        ],
        out_specs=[
            pl.BlockSpec((2, _TR), lambda i: (0, i)),
            pl.BlockSpec((2, _TR), lambda i: (0, i)),
            pl.BlockSpec((_TR, d_in // 2), lambda i: (i, 0)),
        ],
        out_shape=[
            jax.ShapeDtypeStruct((2, t), jnp.int32),
            jax.ShapeDtypeStruct((2, t), jnp.float32),
            jax.ShapeDtypeStruct((t, d_in // 2), jnp.int32),
        ],
    )(x2, gate_wt)
    return ids, ws, xp


def _dispatch(ids, ws, n_e, n_tiles):
    t = ids.shape[1]
    tk = 2 * t
    e_all = ids.reshape(tk)
    e_sorted, order, ws_sorted = lax.sort(
        (e_all, jnp.arange(tk, dtype=jnp.int32), ws.reshape(tk)),
        num_keys=1, is_stable=True)
    idx8 = jnp.arange(n_e, dtype=jnp.int32)
    coff = jnp.searchsorted(e_sorted, idx8, side="left").astype(jnp.int32)
    cend = jnp.searchsorted(e_sorted, idx8, side="right").astype(jnp.int32)
    counts = cend - coff
    padded = ((counts + _TM - 1) // _TM) * _TM
    pcum = jnp.cumsum(padded)
    poff = pcum - padded
    shift = poff - coff
    z8 = jnp.zeros((8,), jnp.int32)
    shift16 = jnp.concatenate([shift, z8])
    cend16 = jnp.concatenate([cend, z8])
    pcum16 = jnp.concatenate([pcum, z8])
    tile_expert = jnp.clip(
        jnp.searchsorted(pcum, jnp.arange(n_tiles, dtype=jnp.int32) * _TM,
                         side="right").astype(jnp.int32), 0, n_e - 1)
    pos_sorted = jnp.arange(tk, dtype=jnp.int32) + shift[e_sorted]
    inv = jnp.zeros((tk,), jnp.int32).at[order].set(pos_sorted)
    return (order, ws_sorted, shift16, cend16, pcum16, tile_expert,
            inv.reshape(2, t))


def _sc_gather(xp, order, ws_flat, shift16, cend16, pcum16, t, ntot):
    info = plsc.get_sparse_core_info()
    nc, ns = info.num_cores, info.num_subcores
    nw = nc * ns
    per_w = ntot // nw
    nit = per_w // _GCH
    tk = 2 * t
    dp = xp.shape[1]
    mesh = plsc.VectorSubcoreMesh(core_axis_name="c", subcore_axis_name="s")

    @functools.partial(
        pl.kernel, mesh=mesh,
        out_type=[
            jax.ShapeDtypeStruct((ntot, dp), jnp.int32),
            jax.ShapeDtypeStruct((ntot,), jnp.float32),
        ],
        scratch_types=[
            pltpu.VMEM((_GCH,), jnp.int32),
            pltpu.VMEM((per_w,), jnp.float32),
            pltpu.VMEM((_GCH, dp), jnp.int32),
            pltpu.VMEM((_GCH, dp), jnp.int32),
            pltpu.VMEM((ntot,), jnp.int32),
            pltpu.VMEM((ntot,), jnp.float32),
            pltpu.VMEM((16,), jnp.int32),
            pltpu.VMEM((16,), jnp.int32),
            pltpu.VMEM((16,), jnp.int32),
            pltpu.SemaphoreType.DMA,
            pltpu.SemaphoreType.DMA,
            pltpu.SemaphoreType.DMA,
        ],
    )
    def k(xp_hbm, order_hbm, ws_hbm, shift_hbm, cend_hbm, pcum_hbm,
          out_hbm, wcol_hbm,
          idx_v, wfull, rows0, rows1, order_v, ws_v, shift_v, cend_v,
          pcum_v, gsem, ssem0, ssem1):
        wid = lax.axis_index("s") * nc + lax.axis_index("c")
        base = wid * per_w
        rows = (rows0, rows1)
        ssem = (ssem0, ssem1)
        pltpu.sync_copy(order_hbm, order_v.at[pl.ds(0, tk)])
        pltpu.sync_copy(ws_hbm, ws_v.at[pl.ds(0, tk)])
        pltpu.sync_copy(shift_hbm, shift_v)
        pltpu.sync_copy(cend_hbm, cend_v)
        pltpu.sync_copy(pcum_hbm, pcum_v)
        lane = lax.iota(jnp.int32, 16)
        sh_vec = shift_v[...]
        ce_vec = cend_v[...]
        pc_vec = pcum_v[...]
        sh_s = [sh_vec[j] for j in range(8)]
        ce_s = [ce_vec[j] for j in range(8)]
        pc_s = [pc_vec[j] for j in range(8)]

        def chunk_meta(p0):
            e0 = jnp.int32(0)
            for j in range(7):
                e0 = e0 + (p0 >= pc_s[j]).astype(jnp.int32)
            sh0 = sh_s[0]
            ce0 = ce_s[0]
            for j in range(1, 8):
                sel = e0 == j
                sh0 = jnp.where(sel, sh_s[j], sh0)
                ce0 = jnp.where(sel, ce_s[j], ce0)
            return p0 - sh0, ce0

        def pair(g, carry):
            for b in range(2):
                it = 2 * g + b
                off = base + it * _GCH
                @pl.when(it >= 2)
                def _drain():
                    pltpu.make_async_copy(
                        rows[b], out_hbm.at[pl.ds(base, _GCH)], ssem[b]).wait()

                for sub in range(_GCH // 16):
                    p0 = off + sub * 16
                    r0, ce0 = chunk_meta(p0)
                    valid = r0 + lane < ce0
                    a = order_v[pl.ds(r0, 16)]
                    tok = jnp.where(valid, a & (t - 1), 0)
                    idx_v[pl.ds(sub * 16, 16)] = tok
                    w = ws_v[pl.ds(r0, 16)]
                    wfull[pl.ds(it * _GCH + sub * 16, 16)] = jnp.where(
                        valid, w, 0.0)

                pltpu.async_copy(xp_hbm.at[idx_v], rows[b], gsem).wait()
                pltpu.async_copy(rows[b], out_hbm.at[pl.ds(off, _GCH)], ssem[b])
            return carry

        lax.fori_loop(0, nit // 2, pair, 0)
        pltpu.sync_copy(wfull, wcol_hbm.at[pl.ds(base, per_w)])
        pltpu.make_async_copy(rows0, out_hbm.at[pl.ds(base, _GCH)], ssem0).wait()
        pltpu.make_async_copy(rows1, out_hbm.at[pl.ds(base, _GCH)], ssem1).wait()

    return k(xp, order, ws_flat, shift16, cend16, pcum16)


def _gmm_body(te_ref, x_ref, w_ref, b_ref, wc_ref, o_ref):
    half = x_ref.shape[1]
    xu = lax.bitcast_convert_type(x_ref[...], jnp.uint32)
    lo = lax.bitcast_convert_type(xu << 16, jnp.float32).astype(jnp.bfloat16)
    hi = lax.bitcast_convert_type(
        xu & jnp.uint32(0xFFFF0000), jnp.float32).astype(jnp.bfloat16)
    w = w_ref[0].astype(jnp.bfloat16)
    dn = (((1,), (1,)), ((), ()))
    y = lax.dot_general(lo, w[:, :half], dn,
                        preferred_element_type=jnp.float32)
    y = y + lax.dot_general(hi, w[:, half:], dn,
                            preferred_element_type=jnp.float32)
    o_ref[...] = (y + b_ref[0]) * wc_ref[...]


def _gmm(xs, tile_expert, expert_w, expert_b3, wcol, n_tiles, d_in, d_out):
    grid = (d_out // _TN, n_tiles)
    return pl.pallas_call(
        _gmm_body,
        grid_spec=pltpu.PrefetchScalarGridSpec(
            num_scalar_prefetch=1,
            grid=grid,
            in_specs=[
                pl.BlockSpec((_TM, d_in // 2), lambda j, i, te: (i, 0)),
                pl.BlockSpec((1, _TN, d_in), lambda j, i, te: (te[i], j, 0)),
                pl.BlockSpec((1, 1, _TN), lambda j, i, te: (te[i], 0, j)),
                pl.BlockSpec((_TM, 1), lambda j, i, te: (i, 0)),
            ],
            out_specs=pl.BlockSpec((_TM, _TN), lambda j, i, te: (i, j)),
        ),
        out_shape=jax.ShapeDtypeStruct((n_tiles * _TM, d_out), jnp.float32),
        compiler_params=pltpu.CompilerParams(
            dimension_semantics=("arbitrary", "arbitrary"),
        ),
    )(tile_expert, xs, expert_w, expert_b3, wcol)


def _sc_combine(ys, inv, t, d_out):
    info = plsc.get_sparse_core_info()
    nc, ns = info.num_cores, info.num_subcores
    nw = nc * ns
    per_w = t // nw
    mesh = plsc.VectorSubcoreMesh(core_axis_name="c", subcore_axis_name="s")
    nsub = d_out // 16

    nit = per_w // _CCH

    @functools.partial(
        pl.kernel, mesh=mesh,
        out_type=jax.ShapeDtypeStruct((t, d_out), jnp.float32),
        scratch_types=[
            pltpu.VMEM((_CCH,), jnp.int32),
            pltpu.VMEM((_CCH,), jnp.int32),
            pltpu.VMEM((_CCH,), jnp.int32),
            pltpu.VMEM((_CCH,), jnp.int32),
            pltpu.VMEM((_CCH, d_out), jnp.float32),
            pltpu.VMEM((_CCH, d_out), jnp.float32),
            pltpu.VMEM((_CCH, d_out), jnp.float32),
            pltpu.VMEM((_CCH, d_out), jnp.float32),
            pltpu.SemaphoreType.DMA,
            pltpu.SemaphoreType.DMA,
            pltpu.SemaphoreType.DMA,
            pltpu.SemaphoreType.DMA,
            pltpu.SemaphoreType.DMA,
            pltpu.SemaphoreType.DMA,
        ],
    )
    def k(ys_hbm, inv_hbm, out_hbm, ia0, ia1, ib0, ib1, a0, b0, a1, b1,
          ga0, gb0, ga1, gb1, ss0, ss1):
        wid = lax.axis_index("s") * nc + lax.axis_index("c")
        base = wid * per_w
        bufa = (a0, a1)
        bufb = (b0, b1)
        gsa = (ga0, ga1)
        gsb = (gb0, gb1)
        ssem = (ss0, ss1)
        idxa = (ia0, ia1)
        idxb = (ib0, ib1)

        def issue(it, s):
            off = base + it * _CCH
            pltpu.sync_copy(inv_hbm.at[0, pl.ds(off, _CCH)], idxa[s])
            pltpu.async_copy(ys_hbm.at[idxa[s]], bufa[s], gsa[s])
            pltpu.sync_copy(inv_hbm.at[1, pl.ds(off, _CCH)], idxb[s])
            pltpu.async_copy(ys_hbm.at[idxb[s]], bufb[s], gsb[s])

        issue(0, 0)

        def pair(g, carry):
            for s in range(2):
                it = 2 * g + s
                nxt = it + 1
                ns_ = 1 - s
                @pl.when(nxt < nit)
                def _issue():
                    @pl.when(nxt >= 2)
                    def _drain():
                        pltpu.make_async_copy(
                            bufa[ns_], out_hbm.at[pl.ds(base, _CCH)],
                            ssem[ns_]).wait()

                    issue(nxt, ns_)

                pltpu.make_async_copy(
                    ys_hbm.at[idxa[s]], bufa[s], gsa[s]).wait()
                pltpu.make_async_copy(
                    ys_hbm.at[idxb[s]], bufb[s], gsb[s]).wait()

                def row(r, c2):
                    for c in range(nsub):
                        sl = pl.ds(c * 16, 16)
                        bufa[s][r, sl] = bufa[s][r, sl] + bufb[s][r, sl]
                    return c2

                lax.fori_loop(0, _CCH, row, 0)
                off = base + it * _CCH
                pltpu.async_copy(bufa[s], out_hbm.at[pl.ds(off, _CCH)], ssem[s])
            return carry

        lax.fori_loop(0, nit // 2, pair, 0)
        pltpu.make_async_copy(a0, out_hbm.at[pl.ds(base, _CCH)], ss0).wait()
        pltpu.make_async_copy(a1, out_hbm.at[pl.ds(base, _CCH)], ss1).wait()

    return k(ys, inv)


@jax.jit
def kernel(inputs, gate_w, expert_w, expert_b):
    b, s, d_in = inputs.shape
    n_e, d_out, _ = expert_w.shape
    t = b * s
    n_tiles = (2 * t) // _TM + n_e
    ntot = n_tiles * _TM

    x2 = inputs.reshape(t, d_in)
    expert_b3 = expert_b.reshape(n_e, 1, d_out)

    ids, ws, xp = _routing(x2, gate_w)
    order, ws_sorted, shift16, cend16, pcum16, tile_expert, inv = _dispatch(
        ids, ws, n_e, n_tiles)
    xsp, wcol_flat = _sc_gather(
        xp, order, ws_sorted, shift16, cend16, pcum16, t, ntot)
    ys = _gmm(xsp, tile_expert, expert_w, expert_b3,
              wcol_flat.reshape(ntot, 1), n_tiles, d_in, d_out)
    out = _sc_combine(ys, inv, t, d_out)
    return out.reshape(b, s, d_out)
